# Initial kernel scaffold; baseline (speedup 1.0000x reference)
import jax
import jax.numpy as jnp
from jax.experimental import pallas as pl


def kernel(x, edge_index, W1, b1, W2, b2, W3, b3):
    raise NotImplementedError("write your pallas kernel here")


# speedup vs baseline: 40.7133x; 40.7133x over previous
import jax
import jax.numpy as jnp
from jax import lax
from jax.experimental import pallas as pl
from jax.experimental.pallas import tpu as pltpu
from jax.experimental.pallas import tpu_sc as plsc

N_NODES = 100000
HIDDEN = 16
N_EDGES = 3200000
K_ITERS = 10
ALPHA = 0.1

NC = 2
NS = 16
NW = NC * NS

CHUNK = 128
BLK = 8
NBLK = 98
RPW = BLK * NBLK
ROWS = RPW * NW
EPAD = ROWS * CHUNK

NPAD = 100352
FLAT = NPAD * HIDDEN
FR = FLAT // 128
TSLC = NPAD // NS
FPT = FR // NS
WCH = 392
NWCH = TSLC // WCH

_mesh = plsc.VectorSubcoreMesh(
    core_axis_name="c", subcore_axis_name="s", num_cores=NC, num_subcores=NS)


def _deg_body(colr_hbm, out0_hbm, out1_hbm, colb, ones, zb, dacc):
    c = lax.axis_index("c")
    s = lax.axis_index("s")
    w = c * NS + s

    def zinit(i, carry):
        zb[pl.ds(i * 16, 16)] = jnp.zeros((16,), jnp.float32)
        return carry

    lax.fori_loop(0, TSLC // 16, zinit, 0)
    for i in range(CHUNK // 16):
        ones[pl.ds(i * 16, 16)] = jnp.ones((16,), jnp.float32)
    pltpu.sync_copy(zb, dacc.at[pl.ds(s * TSLC, TSLC)])
    plsc.subcore_barrier()

    def blk(g, carry):
        base = w * RPW + g * BLK
        pltpu.sync_copy(colr_hbm.at[pl.ds(base, BLK)], colb)
        for j in range(BLK):
            pltpu.sync_copy(ones, dacc.at[colb.at[j]], add=True)
        return carry

    lax.fori_loop(0, NBLK, blk, 0)
    plsc.subcore_barrier()
    sl = pl.ds(s * TSLC, TSLC)
    pltpu.sync_copy(dacc.at[sl], zb)

    @pl.when(c == 0)
    def _():
        pltpu.sync_copy(zb, out0_hbm.at[sl])

    @pl.when(c == 1)
    def _():
        pltpu.sync_copy(zb, out1_hbm.at[sl])


_deg_call = pl.kernel(
    _deg_body,
    out_type=[
        jax.ShapeDtypeStruct((NPAD,), jnp.float32),
        jax.ShapeDtypeStruct((NPAD,), jnp.float32),
    ],
    mesh=_mesh,
    scratch_types=[
        pltpu.VMEM((BLK, CHUNK), jnp.int32),
        pltpu.VMEM((CHUNK,), jnp.float32),
        pltpu.VMEM((TSLC,), jnp.float32),
        pltpu.VMEM_SHARED((NPAD,), jnp.float32),
    ],
    compiler_params=pltpu.CompilerParams(use_tc_tiling_on_sc=False),
)


def _scat_body(rowr_hbm, colr_hbm, y_hbm, out0_hbm, out1_hbm,
               rowb, colb, gbuf, zb, sacc, gsem, ssem):
    c = lax.axis_index("c")
    s = lax.axis_index("s")
    w = c * NS + s

    def zinit(i, carry):
        zb[i, :] = jnp.zeros((16,), jnp.float32)
        return carry

    lax.fori_loop(0, WCH, zinit, 0)
    for i in range(NWCH):
        pltpu.sync_copy(zb, sacc.at[pl.ds(s * TSLC + i * WCH, WCH)])
    plsc.subcore_barrier()

    def blk(g, carry):
        base = w * RPW + g * BLK
        pltpu.sync_copy(rowr_hbm.at[pl.ds(base, BLK)], rowb)
        pltpu.sync_copy(colr_hbm.at[pl.ds(base, BLK)], colb)
        descs = [
            pltpu.async_copy(y_hbm.at[rowb.at[j]], gbuf.at[j], gsem)
            for j in range(BLK)
        ]
        for d in descs:
            d.wait()
        sdescs = [
            pltpu.async_copy(gbuf.at[j], sacc.at[colb.at[j]], ssem, add=True)
            for j in range(BLK)
        ]
        for d in sdescs:
            d.wait()
        return carry

    lax.fori_loop(0, NBLK, blk, 0)
    plsc.subcore_barrier()
    for i in range(NWCH):
        sl = pl.ds(s * TSLC + i * WCH, WCH)
        pltpu.sync_copy(sacc.at[sl], zb)

        @pl.when(c == 0)
        def _():
            pltpu.sync_copy(zb, out0_hbm.at[sl])

        @pl.when(c == 1)
        def _():
            pltpu.sync_copy(zb, out1_hbm.at[sl])


_scat_call = pl.kernel(
    _scat_body,
    out_type=[
        jax.ShapeDtypeStruct((NPAD, HIDDEN), jnp.float32),
        jax.ShapeDtypeStruct((NPAD, HIDDEN), jnp.float32),
    ],
    mesh=_mesh,
    scratch_types=[
        pltpu.VMEM((BLK, CHUNK), jnp.int32),
        pltpu.VMEM((BLK, CHUNK), jnp.int32),
        pltpu.VMEM((BLK, CHUNK, HIDDEN), jnp.float32),
        pltpu.VMEM((WCH, HIDDEN), jnp.float32),
        pltpu.VMEM_SHARED((NPAD, HIDDEN), jnp.float32),
        pltpu.SemaphoreType.DMA,
        pltpu.SemaphoreType.DMA,
    ],
    compiler_params=pltpu.CompilerParams(use_tc_tiling_on_sc=False),
)

RB = 1024
GRID = NPAD // RB
FB = RB * HIDDEN


def _prep_body(x_r, d0_r, d1_r, w1_r, b1_r, w2_r, b2_r,
               y0_r, c1_r, sqd_r):
    x = x_r[...]
    w1 = w1_r[...][:, 0]
    h1 = jnp.maximum(x * w1[None, :] + b1_r[...][None, :], 0.0)
    h = jnp.dot(h1, w2_r[...].T, preferred_element_type=jnp.float32)
    h = jnp.maximum(h + b2_r[...][None, :], 0.0)
    deg = d0_r[...] + d1_r[...] + 1.0
    dinv = lax.rsqrt(deg)
    y0_r[...] = h * dinv[:, None]
    c1_r[...] = jnp.broadcast_to(
        ((1.0 - ALPHA) * dinv * dinv)[:, None], (RB, HIDDEN))
    sqd_r[...] = jnp.sqrt(deg)[:, None]


def _tc_prep(xp, d0, d1, W1, b1, W2, b2):
    f32 = jnp.float32
    return pl.pallas_call(
        _prep_body,
        grid=(GRID,),
        in_specs=[
            pl.BlockSpec((RB, 1), lambda i: (i, 0)),
            pl.BlockSpec((RB,), lambda i: (i,)),
            pl.BlockSpec((RB,), lambda i: (i,)),
            pl.BlockSpec((HIDDEN, 1), lambda i: (0, 0)),
            pl.BlockSpec((HIDDEN,), lambda i: (0,)),
            pl.BlockSpec((HIDDEN, HIDDEN), lambda i: (0, 0)),
            pl.BlockSpec((HIDDEN,), lambda i: (0,)),
        ],
        out_specs=[
            pl.BlockSpec((RB, HIDDEN), lambda i: (i, 0)),
            pl.BlockSpec((RB, HIDDEN), lambda i: (i, 0)),
            pl.BlockSpec((RB, 1), lambda i: (i, 0)),
        ],
        out_shape=[
            jax.ShapeDtypeStruct((NPAD, HIDDEN), f32),
            jax.ShapeDtypeStruct((NPAD, HIDDEN), f32),
            jax.ShapeDtypeStruct((NPAD, 1), f32),
        ],
    )(xp, d0, d1, W1, b1, W2, b2)


def _upd_body(s0_r, s1_r, y_r, a_r, c1_r, o_r):
    o_r[...] = (s0_r[...] + s1_r[...] + y_r[...]) * c1_r[...] + a_r[...]


def _tc_update(S0, S1, y, A, c1):
    spec = pl.BlockSpec((FR // GRID, 128), lambda i: (i, 0))
    return pl.pallas_call(
        _upd_body,
        grid=(GRID,),
        in_specs=[spec, spec, spec, spec, spec],
        out_specs=spec,
        out_shape=jax.ShapeDtypeStruct((FR, 128), jnp.float32),
    )(S0, S1, y, A, c1)


def _fin_body(y_r, sqd_r, w3_r, b3_r, o_r):
    xk = y_r[...] * sqd_r[...]
    o_r[...] = jnp.sum(xk * w3_r[...], axis=1, keepdims=True) + b3_r[...]


def _tc_final(y2d, sqd, W3, b3):
    return pl.pallas_call(
        _fin_body,
        grid=(GRID,),
        in_specs=[
            pl.BlockSpec((RB, HIDDEN), lambda i: (i, 0)),
            pl.BlockSpec((RB, 1), lambda i: (i, 0)),
            pl.BlockSpec((1, HIDDEN), lambda i: (0, 0)),
            pl.BlockSpec((1, 1), lambda i: (0, 0)),
        ],
        out_specs=pl.BlockSpec((RB, 1), lambda i: (i, 0)),
        out_shape=jax.ShapeDtypeStruct((NPAD, 1), jnp.float32),
    )(y2d, sqd, W3, b3)


def kernel(x, edge_index, W1, b1, W2, b2, W3, b3):
    ei = edge_index.astype(jnp.int32)
    npd = EPAD - N_EDGES
    pad_i = lax.iota(jnp.int32, npd)
    rowp = jnp.concatenate([ei[0], pad_i % 4096]).reshape(ROWS, CHUNK)
    colp = jnp.concatenate([ei[1], N_NODES + pad_i % (NPAD - N_NODES)])
    colp = colp.reshape(ROWS, CHUNK)
    xp = jnp.pad(x, ((0, NPAD - N_NODES), (0, 0)))

    d0, d1 = _deg_call(colp)
    y0, c1, sqd = _tc_prep(xp, d0, d1, W1, b1, W2, b2)
    y = y0
    A = ALPHA * y0.reshape(FR, 128)
    c1 = c1.reshape(FR, 128)
    for _ in range(K_ITERS):
        S0, S1 = _scat_call(rowp, colp, y)
        yf = _tc_update(S0.reshape(FR, 128), S1.reshape(FR, 128),
                        y.reshape(FR, 128), A, c1)
        y = yf.reshape(NPAD, HIDDEN)
    out = _tc_final(y, sqd, W3, b3.reshape(1, 1))
    return out[:N_NODES]
